# Initial kernel scaffold; baseline (speedup 1.0000x reference)
import jax
import jax.numpy as jnp
from jax.experimental import pallas as pl


def kernel(x, time_day, time_week):
    raise NotImplementedError("write your pallas kernel here")


# speedup vs baseline: 3.7691x; 3.7691x over previous
import functools

import jax
import jax.numpy as jnp
from jax import lax
from jax.experimental import pallas as pl
from jax.experimental.pallas import tpu as pltpu
from jax.experimental.pallas import tpu_sc as plsc

_TIME = 288
_NWEEK = 7
_B = 64
_T = 12
_N = 4096
_F = 64

_CHUNK = 512
_CPB = _N // _CHUNK
_NCHUNKS = _B * _CPB

_info = plsc.get_sparse_core_info()
_NC = _info.num_cores
_NS = _info.num_subcores
_NW = _NC * _NS
_CPW = _NCHUNKS // _NW
_L = 16


@functools.partial(
    pl.kernel,
    mesh=plsc.VectorSubcoreMesh(core_axis_name="c", subcore_axis_name="s"),
    compiler_params=pltpu.CompilerParams(needs_layout_passes=False),
    out_type=jax.ShapeDtypeStruct((_B, _F, _N), jnp.float32),
    scratch_types=[
        pltpu.VMEM((_F * _TIME,), jnp.float32),
        pltpu.VMEM((_F * _NWEEK,), jnp.float32),
        pltpu.VMEM((_CHUNK * 3,), jnp.float32),
        pltpu.VMEM((_F, _CHUNK), jnp.float32),
    ],
)
def _emb_kernel(x_hbm, td_hbm, tw_hbm, out_hbm, td_v, tw_v, xbuf, blk):
    wid = lax.axis_index("s") * _NC + lax.axis_index("c")

    pltpu.sync_copy(td_hbm, td_v)
    pltpu.sync_copy(tw_hbm, tw_v)

    lane3 = lax.iota(jnp.int32, _L) * 3

    def chunk_body(i, carry):
        cid = wid * _CPW + i
        b = cid // _CPB
        n0 = (cid % _CPB) * _CHUNK

        pltpu.sync_copy(x_hbm.at[b, _T - 1, pl.ds(n0 * 3, _CHUNK * 3)], xbuf)

        def v_body(v, c):
            tok3 = v * (_L * 3) + lane3
            dayv = plsc.load_gather(xbuf, [tok3 + 1])
            weekv = plsc.load_gather(xbuf, [tok3 + 2])
            day_i = jnp.clip((dayv * float(_TIME)).astype(jnp.int32),
                             0, _TIME - 1)
            week_i = jnp.clip(weekv.astype(jnp.int32), 0, _NWEEK - 1)
            for f in range(_F):
                d = plsc.load_gather(td_v, [day_i + f * _TIME])
                w = plsc.load_gather(tw_v, [week_i + f * _NWEEK])
                blk[f, pl.ds(v * _L, _L)] = d + w
            return c

        lax.fori_loop(0, _CHUNK // _L, v_body, 0)
        pltpu.sync_copy(blk, out_hbm.at[b, :, pl.ds(n0, _CHUNK)])
        return carry

    lax.fori_loop(0, _CPW, chunk_body, 0)


def kernel(x, time_day, time_week):
    td = time_day.T.reshape(-1)
    tw = time_week.T.reshape(-1)
    out = _emb_kernel(x.reshape(_B, _T, _N * 3), td, tw)
    return out[..., None]
